# Initial kernel scaffold; baseline (speedup 1.0000x reference)
import jax
import jax.numpy as jnp
from jax.experimental import pallas as pl


def kernel(token_ids, token_type_ids, field_ids, entity_ids, time_ids, token_table, pos_table, type_table, field_table, entity_table, time_table):
    raise NotImplementedError("write your pallas kernel here")


# speedup vs baseline: 1.2407x; 1.2407x over previous
import functools

import jax
import jax.numpy as jnp
from jax import lax
from jax.experimental import pallas as pl
from jax.experimental.pallas import tpu as pltpu
from jax.experimental.pallas import tpu_sc as plsc

B, S, D = 1024, 512, 64
N = B * S

_info = plsc.get_sparse_core_info()
NC, NS, L = _info.num_cores, _info.num_subcores, _info.num_lanes
NW = NC * NS
PER_W = N // NW
K = 128
CHUNKS = PER_W // K

_mesh = plsc.VectorSubcoreMesh(core_axis_name="c", subcore_axis_name="s")


@functools.partial(
    pl.kernel,
    mesh=_mesh,
    compiler_params=pltpu.CompilerParams(use_tc_tiling_on_sc=False),
    out_type=jax.ShapeDtypeStruct((N, D), jnp.float32),
    scratch_types=[
        pltpu.VMEM((K,), jnp.int32),
        pltpu.VMEM((K,), jnp.int32),
        pltpu.VMEM((K,), jnp.int32),
        pltpu.VMEM((K,), jnp.int32),
        pltpu.VMEM((K,), jnp.int32),
        pltpu.VMEM((K, D), jnp.float32),
        pltpu.VMEM((K, D), jnp.float32),
        pltpu.VMEM((K, D), jnp.float32),
        pltpu.VMEM((K, D), jnp.float32),
        pltpu.VMEM((K, D), jnp.float32),
        pltpu.VMEM((S, D), jnp.float32),
        pltpu.SemaphoreType.DMA,
    ],
)
def _emb_kernel(tok_i, typ_i, fld_i, ent_i, tim_i,
                tok_t, pos_t, typ_t, fld_t, ent_t, tim_t,
                out,
                tok_iv, typ_iv, fld_iv, ent_iv, tim_iv,
                tok_v, typ_v, fld_v, ent_v, tim_v, pos_v, sem):
    wid = lax.axis_index("s") * NC + lax.axis_index("c")
    base0 = wid * PER_W
    pltpu.sync_copy(pos_t, pos_v)

    def chunk_body(c, carry):
        base = base0 + c * K
        i1 = pltpu.async_copy(tok_i.at[pl.ds(base, K)], tok_iv, sem)
        i2 = pltpu.async_copy(typ_i.at[pl.ds(base, K)], typ_iv, sem)
        i3 = pltpu.async_copy(fld_i.at[pl.ds(base, K)], fld_iv, sem)
        i4 = pltpu.async_copy(ent_i.at[pl.ds(base, K)], ent_iv, sem)
        i5 = pltpu.async_copy(tim_i.at[pl.ds(base, K)], tim_iv, sem)
        i1.wait(); i2.wait(); i3.wait(); i4.wait(); i5.wait()
        g1 = pltpu.async_copy(tok_t.at[tok_iv], tok_v, sem)
        g2 = pltpu.async_copy(typ_t.at[typ_iv], typ_v, sem)
        g3 = pltpu.async_copy(fld_t.at[fld_iv], fld_v, sem)
        g4 = pltpu.async_copy(ent_t.at[ent_iv], ent_v, sem)
        g5 = pltpu.async_copy(tim_t.at[tim_iv], tim_v, sem)
        g1.wait(); g2.wait(); g3.wait(); g4.wait(); g5.wait()

        pos_off = (c * K) % S

        def row_body(p, carry2):
            for k in range(D // L):
                sl = pl.ds(k * L, L)
                acc = (tok_v[p, sl] + typ_v[p, sl] + fld_v[p, sl]
                       + ent_v[p, sl] + tim_v[p, sl] + pos_v[pos_off + p, sl])
                tok_v[p, sl] = acc
            return carry2

        lax.fori_loop(0, K, row_body, 0)
        pltpu.sync_copy(tok_v, out.at[pl.ds(base, K)])
        return carry

    lax.fori_loop(0, CHUNKS, chunk_body, 0)


def kernel(token_ids, token_type_ids, field_ids, entity_ids, time_ids,
           token_table, pos_table, type_table, field_table, entity_table, time_table):
    tok = token_ids.reshape(-1).astype(jnp.int32)
    typ = token_type_ids.reshape(-1).astype(jnp.int32)
    fld = field_ids.reshape(-1).astype(jnp.int32)
    ent = entity_ids.reshape(-1).astype(jnp.int32)
    tim = time_ids.reshape(-1).astype(jnp.int32)
    out = _emb_kernel(tok, typ, fld, ent, tim,
                      token_table, pos_table, type_table,
                      field_table, entity_table, time_table)
    return out.reshape(B, S, D)


# speedup vs baseline: 1.2439x; 1.0026x over previous
import functools

import jax
import jax.numpy as jnp
from jax import lax
from jax.experimental import pallas as pl
from jax.experimental.pallas import tpu as pltpu
from jax.experimental.pallas import tpu_sc as plsc

B, S, D = 1024, 512, 64
N = B * S

_info = plsc.get_sparse_core_info()
NC, NS, L = _info.num_cores, _info.num_subcores, _info.num_lanes
NW = NC * NS
PER_W = N // NW
K = 128
CHUNKS = PER_W // K
NBUF = 2

_mesh = plsc.VectorSubcoreMesh(core_axis_name="c", subcore_axis_name="s")

_scratch = (
    [pltpu.VMEM((K,), jnp.int32) for _ in range(5 * NBUF)]
    + [pltpu.VMEM((K, D), jnp.float32) for _ in range(5 * NBUF)]
    + [
        pltpu.VMEM((K, D), jnp.float32),
        pltpu.VMEM((S, D), jnp.float32),
        pltpu.SemaphoreType.DMA,
        pltpu.SemaphoreType.DMA,
        pltpu.SemaphoreType.DMA,
        pltpu.SemaphoreType.DMA,
        pltpu.SemaphoreType.DMA,
    ]
)


@functools.partial(
    pl.kernel,
    mesh=_mesh,
    compiler_params=pltpu.CompilerParams(use_tc_tiling_on_sc=False),
    out_type=jax.ShapeDtypeStruct((N, D), jnp.float32),
    scratch_types=_scratch,
)
def _emb_kernel(tok_i, typ_i, fld_i, ent_i, tim_i,
                tok_t, pos_t, typ_t, fld_t, ent_t, tim_t,
                out, *scr):
    idx_hbm = [tok_i, typ_i, fld_i, ent_i, tim_i]
    tbl_hbm = [tok_t, typ_t, fld_t, ent_t, tim_t]
    iv = [scr[0:5], scr[5:10]]
    rv = [scr[10:15], scr[15:20]]
    out_v = scr[20]
    pos_v = scr[21]
    sem_i = [scr[22], scr[23]]
    sem_g = [scr[24], scr[25]]
    sem_st = scr[26]

    wid = lax.axis_index("s") * NC + lax.axis_index("c")
    base0 = wid * PER_W

    def issue_idx(c, b):
        base = base0 + c * K
        for h, v in zip(idx_hbm, iv[b]):
            pltpu.async_copy(h.at[pl.ds(base, K)], v, sem_i[b])

    def wait_idx(b):
        for h, v in zip(idx_hbm, iv[b]):
            pltpu.make_async_copy(h.at[pl.ds(0, K)], v, sem_i[b]).wait()

    def issue_gathers(c, b):
        for t, ix, v in zip(tbl_hbm, iv[b], rv[b]):
            pltpu.async_copy(t.at[ix], v, sem_g[b])

    def wait_gathers(b):
        for t, ix, v in zip(tbl_hbm, iv[b], rv[b]):
            pltpu.make_async_copy(t.at[ix], v, sem_g[b]).wait()

    def wait_store():
        pltpu.make_async_copy(out_v, out.at[pl.ds(base0, K)], sem_st).wait()

    def compute(c, b):
        pos_off = lax.rem(c, S // K) * K
        tok_v, typ_v, fld_v, ent_v, tim_v = rv[b]

        def row_body(p, carry):
            for k in range(D // L):
                sl = pl.ds(k * L, L)
                out_v[p, sl] = (tok_v[p, sl] + typ_v[p, sl] + fld_v[p, sl]
                                + ent_v[p, sl] + tim_v[p, sl]
                                + pos_v[pos_off + p, sl])
            return carry

        lax.fori_loop(0, K, row_body, 0)

    pltpu.sync_copy(pos_t, pos_v)
    issue_idx(0, 0)
    issue_idx(1, 1)
    wait_idx(0)
    issue_gathers(0, 0)

    def super_body(cc, carry):
        for b in range(NBUF):
            c = cc * NBUF + b
            nb = 1 - b

            @pl.when(c + 1 < CHUNKS)
            def _():
                wait_idx(nb)
                issue_gathers(c + 1, nb)

            wait_gathers(b)

            @pl.when(c + 2 < CHUNKS)
            def _():
                issue_idx(c + 2, b)

            @pl.when(c >= 1)
            def _():
                wait_store()

            compute(c, b)
            pltpu.async_copy(out_v, out.at[pl.ds(base0 + c * K, K)], sem_st)
        return carry

    lax.fori_loop(0, CHUNKS // NBUF, super_body, 0)
    wait_store()


def kernel(token_ids, token_type_ids, field_ids, entity_ids, time_ids,
           token_table, pos_table, type_table, field_table, entity_table, time_table):
    tok = token_ids.reshape(-1).astype(jnp.int32)
    typ = token_type_ids.reshape(-1).astype(jnp.int32)
    fld = field_ids.reshape(-1).astype(jnp.int32)
    ent = entity_ids.reshape(-1).astype(jnp.int32)
    tim = time_ids.reshape(-1).astype(jnp.int32)
    out = _emb_kernel(tok, typ, fld, ent, tim,
                      token_table, pos_table, type_table,
                      field_table, entity_table, time_table)
    return out.reshape(B, S, D)
